# baseline (speedup 1.0000x reference)
import functools

import jax
import jax.numpy as jnp
from jax import lax
from jax.experimental import pallas as pl
from jax.experimental.pallas import tpu as pltpu
from jax.experimental.pallas import tpu_sc as plsc

_BLOCK = 1000
_NSEL = 64
_STRIDE = 1000
_COLS = 512
_L = 16
_NC = 2
_NS = 16


def _copy_body(x_ref, o_ref):
    o_ref[...] = x_ref[...]


def _sc_fix_body(xref, buf):
    c = lax.axis_index("c")
    s = lax.axis_index("s")
    w = s * _NC + c
    for j in range(2):
        row = (2 * w + j) * _STRIDE
        pltpu.sync_copy(xref.at[row], buf.at[j])
        for t in range(_COLS // _L):
            v = buf[j, pl.ds(t * _L, _L)]
            buf[j, pl.ds(t * _L, _L)] = jnp.maximum(v, 0.0)
        pltpu.sync_copy(buf.at[j], xref.at[row])


@functools.cache
def _sc_fix():
    mesh = plsc.VectorSubcoreMesh(
        core_axis_name="c", subcore_axis_name="s", num_cores=_NC, num_subcores=_NS
    )
    return pl.kernel(
        _sc_fix_body,
        out_type=(),
        mesh=mesh,
        scratch_types=[pltpu.VMEM((2, _COLS), jnp.float32)],
    )


def kernel(x):
    rows, cols = x.shape
    y = pl.pallas_call(
        _copy_body,
        grid=(rows // _BLOCK,),
        in_specs=[pl.BlockSpec((_BLOCK, cols), lambda i: (i, 0))],
        out_specs=pl.BlockSpec((_BLOCK, cols), lambda i: (i, 0)),
        out_shape=jax.ShapeDtypeStruct(x.shape, x.dtype),
    )(x)
    ref = jax.new_ref(y)
    _sc_fix()(ref)
    return jax.freeze(ref)


# speedup vs baseline: 1.2551x; 1.2551x over previous
import jax
import jax.numpy as jnp
from jax.experimental import pallas as pl
from jax.experimental.pallas import tpu as pltpu

_CHUNK = 2000
_NB = 10
_K = 5
_NSEL = 64
_T = 8


def _body(x_hbm, o_hbm, bufs, insem, outsem):
    rows, cols = x_hbm.shape
    nchunks = rows // _CHUNK

    def in_copy(c):
        return pltpu.make_async_copy(
            x_hbm.at[pl.ds(c * _CHUNK, _CHUNK), :],
            bufs.at[c % _NB],
            insem.at[c % _NB],
        )

    def out_copy(c):
        return pltpu.make_async_copy(
            bufs.at[c % _NB],
            o_hbm.at[pl.ds(c * _CHUNK, _CHUNK), :],
            outsem.at[c % _NB],
        )

    for c in range(_K):
        in_copy(c).start()
    for c in range(nchunks):
        in_copy(c).wait()
        b = c % _NB
        for j in range(_CHUNK // 1000):
            if c * (_CHUNK // 1000) + j < _NSEL:
                r = j * 1000
                slab = bufs[b, r:r + _T, :]
                rid = jax.lax.broadcasted_iota(jnp.int32, slab.shape, 0)
                bufs[b, r:r + _T, :] = jnp.where(
                    rid == 0, jnp.maximum(slab, 0.0), slab
                )
        out_copy(c).start()
        nxt = c + _K
        if nxt < nchunks:
            prev = nxt - _NB
            if prev >= 0:
                out_copy(prev).wait()
            in_copy(nxt).start()
    for c in range(max(nchunks - _NB, 0), nchunks):
        out_copy(c).wait()


def kernel(x):
    rows, cols = x.shape
    return pl.pallas_call(
        _body,
        in_specs=[pl.BlockSpec(memory_space=pltpu.HBM)],
        out_specs=pl.BlockSpec(memory_space=pltpu.HBM),
        out_shape=jax.ShapeDtypeStruct(x.shape, x.dtype),
        scratch_shapes=[
            pltpu.VMEM((_NB, _CHUNK, cols), jnp.float32),
            pltpu.SemaphoreType.DMA((_NB,)),
            pltpu.SemaphoreType.DMA((_NB,)),
        ],
        compiler_params=pltpu.CompilerParams(vmem_limit_bytes=100 * 1024 * 1024),
    )(x)


# speedup vs baseline: 1.2575x; 1.0019x over previous
import jax
import jax.numpy as jnp
from jax.experimental import pallas as pl
from jax.experimental.pallas import tpu as pltpu

_BLOCK = 5000
_PER = 5
_NSEL = 64
_T = 8


def _body(x_ref, o_ref):
    i = pl.program_id(0)
    o_ref[...] = x_ref[...]
    for j in range(_PER):
        @pl.when(i * _PER + j < _NSEL)
        def _fix(j=j):
            r = j * 1000
            slab = x_ref[r:r + _T, :]
            rid = jax.lax.broadcasted_iota(jnp.int32, slab.shape, 0)
            o_ref[r:r + _T, :] = jnp.where(rid == 0, jnp.maximum(slab, 0.0), slab)


def kernel(x):
    rows, cols = x.shape
    grid = rows // _BLOCK
    return pl.pallas_call(
        _body,
        grid=(grid,),
        in_specs=[pl.BlockSpec((_BLOCK, cols), lambda i: (i, 0))],
        out_specs=pl.BlockSpec((_BLOCK, cols), lambda i: (i, 0)),
        out_shape=jax.ShapeDtypeStruct(x.shape, x.dtype),
        compiler_params=pltpu.CompilerParams(vmem_limit_bytes=100 * 1024 * 1024),
    )(x)
